# baseline (speedup 1.0000x reference)
import jax, jax.numpy as jnp
from jax.experimental import pallas as pl

def kernel(node_attrs, node_feats, edge_attrs, edge_feats, edge_index, W_up, W_mlp0, W_mlp1, W_mlp2, W_mlp3, W_lin_s, W_lin_v, W_skip):
    sender = edge_index[0]
    receiver = edge_index[1]
    num_nodes = node_feats.shape[0]
    x = node_feats @ W_up
    h = jax.nn.silu(edge_feats @ W_mlp0)
    h = jax.nn.silu(h @ W_mlp1)
    h = jax.nn.silu(h @ W_mlp2)
    tp_w = h @ W_mlp3
    w0 = tp_w[:, :64]
    w1 = tp_w[:, 64:]
    xs = x[sender]
    sh0 = edge_attrs[:, 0:1]
    sh1 = edge_attrs[:, 1:4]
    m0 = w0 * xs * sh0
    m1 = (w1 * xs)[:, :, None] * sh1[:, None, :]
    msg0 = jnp.zeros((num_nodes, 64), dtype=m0.dtype).at[receiver].add(m0)
    msg1 = jnp.zeros((num_nodes, 64, 3), dtype=m1.dtype).at[receiver].add(m1)
    out0 = (msg0 @ W_lin_s) / 16.0
    out1 = jnp.einsum('num,uv->nvm', msg1, W_lin_v) / 16.0
    outer = (x[:, :, None] * node_attrs[:, None, :]).reshape(num_nodes, 640)
    sc0 = outer @ W_skip
    return jnp.concatenate([out0 + sc0, out1.reshape(num_nodes, 192)], axis=1)


# speedup vs baseline: 8.7438x; 8.7438x over previous
import functools

import jax
import jax.numpy as jnp
from jax import lax
from jax.experimental import pallas as pl
from jax.experimental.pallas import tpu as pltpu
from jax.experimental.pallas import tpu_sc as plsc

N = 50000
E = 800000
NUM_ELEM = 10
LATENT = 64
RADIAL = 8
INV_AVG = 1.0 / 16.0

NWORKERS = 32
WINDOW = 368
ACCROWS = 376
ROUNDS = 5
OUTROWS = ROUNDS * NWORKERS * WINDOW
STRIP = 800
NSTRIPS = E // STRIP
B = 64
CAP = STRIP + 128
WROW = 4 * LATENT
XROW = 2 * LATENT


def _silu(v):
    return v * jax.nn.sigmoid(v)


def _node_up_body(nf_ref, na_ref, wup_ref, wsk_ref, x_ref, sc0_ref):
    x = jnp.dot(nf_ref[...], wup_ref[...], preferred_element_type=jnp.float32)
    x_ref[...] = jnp.concatenate(
        [x, jnp.zeros((x.shape[0], XROW - LATENT), jnp.float32)], axis=1)
    na = na_ref[...]
    acc = jnp.zeros_like(x)
    for k in range(NUM_ELEM):
        acc = acc + jnp.dot(x * na[:, k:k + 1], wsk_ref[:, k, :],
                            preferred_element_type=jnp.float32)
    sc0_ref[...] = acc


def _edge_mlp_body(ef_ref, ea_ref, w0_ref, w1_ref, w2_ref, w3_ref, out_ref):
    h = _silu(jnp.dot(ef_ref[...], w0_ref[...], preferred_element_type=jnp.float32))
    h = _silu(jnp.dot(h, w1_ref[...], preferred_element_type=jnp.float32))
    h = _silu(jnp.dot(h, w2_ref[...], preferred_element_type=jnp.float32))
    tpw = jnp.dot(h, w3_ref[...], preferred_element_type=jnp.float32)
    ea = ea_ref[...]
    w0s = tpw[:, :LATENT] * ea[:, 0:1]
    w1 = tpw[:, LATENT:]
    out_ref[...] = jnp.concatenate(
        [w0s, w1 * ea[:, 1:2], w1 * ea[:, 2:3], w1 * ea[:, 3:4]], axis=1)


def _sc_body(x_hbm, w_hbm, send_hbm, recv_hbm, out_hbm,
             rbuf0, sbuf0, rbuf1, sbuf1, cid, csend, crecv, xg, wg, acc,
             semr0, sems0, semr1, sems1, semx, semw):
    core = lax.axis_index("c")
    tid = lax.axis_index("s")
    wid = tid * 2 + core
    iota16 = lax.iota(jnp.int32, 16)

    def start_strip(s, rb, sb, semr, sems):
        pltpu.make_async_copy(
            recv_hbm.at[pl.ds(s * STRIP, STRIP)], rb, semr).start()
        pltpu.make_async_copy(
            send_hbm.at[pl.ds(s * STRIP, STRIP)], sb, sems).start()

    def wait_strip(rb, sb, semr, sems):
        pltpu.make_async_copy(recv_hbm.at[pl.ds(0, STRIP)], rb, semr).wait()
        pltpu.make_async_copy(send_hbm.at[pl.ds(0, STRIP)], sb, sems).wait()

    def process_batch(off):
        cp1 = pltpu.async_copy(x_hbm.at[csend.at[pl.ds(off, B)]], xg, semx)
        cp2 = pltpu.async_copy(w_hbm.at[cid.at[pl.ds(off, B)]], wg, semw)
        cp1.wait()
        cp2.wait()

        def grp_body(g, _):
            rv = crecv[pl.ds(off + 16 * g, 16)]
            for j in range(16):
                e = 16 * g + j
                row = rv[j]
                for k in range(4):
                    xk = xg[e, pl.ds(16 * k, 16)]
                    for m in range(4):
                        plsc.addupdate(
                            acc.at[row, pl.ds(LATENT * m + 16 * k, 16)],
                            wg[e, pl.ds(LATENT * m + 16 * k, 16)] * xk)
            return 0
        lax.fori_loop(0, B // 16, grp_body, 0)

    def scan_strip(s, cnt, lo, rb, sb):
        sbase = s * STRIP

        def scan_body(i, cnt):
            r = rb[pl.ds(i * 16, 16)]
            rloc = r - lo
            m = (rloc >= 0) & (rloc < WINDOW)
            c16 = plsc.all_reduce_population_count(m)
            c0 = c16[0]

            @pl.when(c0 > 0)
            def _():
                sv = sb[pl.ds(i * 16, 16)]
                mi = m.astype(jnp.int32)
                pos = cnt + plsc.cumsum(mi) - 1
                eid = sbase + i * 16 + iota16
                plsc.store_scatter(cid, [pos], eid, mask=m)
                plsc.store_scatter(csend, [pos], sv, mask=m)
                plsc.store_scatter(crecv, [pos], rloc, mask=m)
            return cnt + c0
        cnt = lax.fori_loop(0, STRIP // 16, scan_body, cnt)

        nb = cnt // B

        @pl.when(nb > 0)
        def _():
            def b_body(b, _):
                process_batch(b * B)
                return 0
            lax.fori_loop(0, nb, b_body, 0)

            def mv(j, _):
                src = nb * B + j * 16
                cid[pl.ds(j * 16, 16)] = cid[pl.ds(src, 16)]
                csend[pl.ds(j * 16, 16)] = csend[pl.ds(src, 16)]
                crecv[pl.ds(j * 16, 16)] = crecv[pl.ds(src, 16)]
                return 0
            lax.fori_loop(0, B // 16, mv, 0)
        return cnt - nb * B

    def round_body(rnd, _):
        win = rnd * NWORKERS + wid
        lo = win * WINDOW

        def zr(i, _):
            for s in range(16):
                acc[i, pl.ds(16 * s, 16)] = jnp.zeros((16,), jnp.float32)
            return 0
        lax.fori_loop(0, ACCROWS, zr, 0)

        start_strip(0, rbuf0, sbuf0, semr0, sems0)

        def pair_body(p, cnt):
            start_strip(2 * p + 1, rbuf1, sbuf1, semr1, sems1)
            wait_strip(rbuf0, sbuf0, semr0, sems0)
            cnt = scan_strip(2 * p, cnt, lo, rbuf0, sbuf0)

            @pl.when(p + 1 < NSTRIPS // 2)
            def _():
                start_strip(2 * p + 2, rbuf0, sbuf0, semr0, sems0)
            wait_strip(rbuf1, sbuf1, semr1, sems1)
            cnt = scan_strip(2 * p + 1, cnt, lo, rbuf1, sbuf1)
            return cnt
        cnt = lax.fori_loop(0, NSTRIPS // 2, pair_body, jnp.int32(0))

        def padj(j, _):
            lane = j * 16 + iota16
            keep = lane < cnt
            crecv[pl.ds(j * 16, 16)] = jnp.where(
                keep, crecv[pl.ds(j * 16, 16)], WINDOW)
            csend[pl.ds(j * 16, 16)] = jnp.where(
                keep, csend[pl.ds(j * 16, 16)], iota16)
            cid[pl.ds(j * 16, 16)] = jnp.where(
                keep, cid[pl.ds(j * 16, 16)], iota16)
            return 0
        lax.fori_loop(0, B // 16, padj, 0)
        process_batch(0)

        pltpu.sync_copy(acc.at[pl.ds(0, WINDOW)],
                        out_hbm.at[pl.ds(lo, WINDOW)])
        return 0

    lax.fori_loop(0, ROUNDS, round_body, 0)


def _out_body(msg_ref, sc0_ref, wls_ref, wbig_ref, out_ref):
    msg = msg_ref[...]
    o0 = jnp.dot(msg[:, :LATENT], wls_ref[...],
                 preferred_element_type=jnp.float32) + sc0_ref[...]
    o1 = jnp.dot(msg[:, LATENT:], wbig_ref[...],
                 preferred_element_type=jnp.float32)
    out_ref[...] = jnp.concatenate([o0, o1], axis=1)


def kernel(node_attrs, node_feats, edge_attrs, edge_feats, edge_index,
           W_up, W_mlp0, W_mlp1, W_mlp2, W_mlp3, W_lin_s, W_lin_v, W_skip):
    sender = edge_index[0]
    receiver = edge_index[1]
    wsk = W_skip.reshape(LATENT, NUM_ELEM, LATENT)

    NB = 2000
    x, sc0 = pl.pallas_call(
        _node_up_body,
        grid=(N // NB,),
        in_specs=[
            pl.BlockSpec((NB, LATENT), lambda i: (i, 0)),
            pl.BlockSpec((NB, NUM_ELEM), lambda i: (i, 0)),
            pl.BlockSpec((LATENT, LATENT), lambda i: (0, 0)),
            pl.BlockSpec((LATENT, NUM_ELEM, LATENT), lambda i: (0, 0, 0)),
        ],
        out_specs=[
            pl.BlockSpec((NB, XROW), lambda i: (i, 0)),
            pl.BlockSpec((NB, LATENT), lambda i: (i, 0)),
        ],
        out_shape=[
            jax.ShapeDtypeStruct((N, XROW), jnp.float32),
            jax.ShapeDtypeStruct((N, LATENT), jnp.float32),
        ],
    )(node_feats, node_attrs, W_up, wsk)

    EB = 2000
    wfold = pl.pallas_call(
        _edge_mlp_body,
        grid=(E // EB,),
        in_specs=[
            pl.BlockSpec((EB, RADIAL), lambda i: (i, 0)),
            pl.BlockSpec((EB, 4), lambda i: (i, 0)),
            pl.BlockSpec((RADIAL, 64), lambda i: (0, 0)),
            pl.BlockSpec((64, 64), lambda i: (0, 0)),
            pl.BlockSpec((64, 64), lambda i: (0, 0)),
            pl.BlockSpec((64, 2 * LATENT), lambda i: (0, 0)),
        ],
        out_specs=pl.BlockSpec((EB, WROW), lambda i: (i, 0)),
        out_shape=jax.ShapeDtypeStruct((E, WROW), jnp.float32),
    )(edge_feats, edge_attrs, W_mlp0, W_mlp1, W_mlp2, W_mlp3)

    mesh = plsc.VectorSubcoreMesh(core_axis_name="c", subcore_axis_name="s")
    sc_call = functools.partial(
        pl.kernel,
        mesh=mesh,
        out_type=jax.ShapeDtypeStruct((OUTROWS, 4 * LATENT), jnp.float32),
        compiler_params=pltpu.CompilerParams(needs_layout_passes=False),
        scratch_types=[
            pltpu.VMEM((STRIP,), jnp.int32),
            pltpu.VMEM((STRIP,), jnp.int32),
            pltpu.VMEM((STRIP,), jnp.int32),
            pltpu.VMEM((STRIP,), jnp.int32),
            pltpu.VMEM((CAP,), jnp.int32),
            pltpu.VMEM((CAP,), jnp.int32),
            pltpu.VMEM((CAP,), jnp.int32),
            pltpu.VMEM((B, XROW), jnp.float32),
            pltpu.VMEM((B, WROW), jnp.float32),
            pltpu.VMEM((ACCROWS, WROW), jnp.float32),
            pltpu.SemaphoreType.DMA,
            pltpu.SemaphoreType.DMA,
            pltpu.SemaphoreType.DMA,
            pltpu.SemaphoreType.DMA,
            pltpu.SemaphoreType.DMA,
            pltpu.SemaphoreType.DMA,
        ],
    )(_sc_body)
    msgpad = sc_call(x, wfold, sender, receiver)

    wls = W_lin_s * INV_AVG
    wbig = ((W_lin_v[None, :, :, None] * jnp.eye(3, dtype=jnp.float32)[:, None, None, :])
            .reshape(3 * LATENT, 3 * LATENT) * INV_AVG)
    out = pl.pallas_call(
        _out_body,
        grid=(N // NB,),
        in_specs=[
            pl.BlockSpec((NB, 4 * LATENT), lambda i: (i, 0)),
            pl.BlockSpec((NB, LATENT), lambda i: (i, 0)),
            pl.BlockSpec((LATENT, LATENT), lambda i: (0, 0)),
            pl.BlockSpec((3 * LATENT, 3 * LATENT), lambda i: (0, 0)),
        ],
        out_specs=pl.BlockSpec((NB, 4 * LATENT), lambda i: (i, 0)),
        out_shape=jax.ShapeDtypeStruct((N, 4 * LATENT), jnp.float32),
    )(msgpad, sc0, wls, wbig)
    return out


# speedup vs baseline: 8.9312x; 1.0214x over previous
import functools

import jax
import jax.numpy as jnp
from jax import lax
from jax.experimental import pallas as pl
from jax.experimental.pallas import tpu as pltpu
from jax.experimental.pallas import tpu_sc as plsc

N = 50000
E = 800000
NUM_ELEM = 10
LATENT = 64
RADIAL = 8
INV_AVG = 1.0 / 16.0

NWORKERS = 32
WINDOW = 344
ACCROWS = 352
ROUNDS = 5
OUTROWS = ROUNDS * NWORKERS * WINDOW
STRIP = 2000
NSTRIPS = E // STRIP
B = 64
CAP = STRIP + 128
WROW = 4 * LATENT
XROW = 2 * LATENT


def _silu(v):
    return v * jax.nn.sigmoid(v)


def _node_up_body(nf_ref, na_ref, wup_ref, wsk_ref, x_ref, sc0_ref):
    x = jnp.dot(nf_ref[...], wup_ref[...], preferred_element_type=jnp.float32)
    x_ref[...] = jnp.concatenate(
        [x, jnp.zeros((x.shape[0], XROW - LATENT), jnp.float32)], axis=1)
    na = na_ref[...]
    acc = jnp.zeros_like(x)
    for k in range(NUM_ELEM):
        acc = acc + jnp.dot(x * na[:, k:k + 1], wsk_ref[:, k, :],
                            preferred_element_type=jnp.float32)
    sc0_ref[...] = acc


def _edge_mlp_body(ef_ref, ea_ref, w0_ref, w1_ref, w2_ref, w3_ref, out_ref):
    h = _silu(jnp.dot(ef_ref[...], w0_ref[...], preferred_element_type=jnp.float32))
    h = _silu(jnp.dot(h, w1_ref[...], preferred_element_type=jnp.float32))
    h = _silu(jnp.dot(h, w2_ref[...], preferred_element_type=jnp.float32))
    tpw = jnp.dot(h, w3_ref[...], preferred_element_type=jnp.float32)
    ea = ea_ref[...]
    w0s = tpw[:, :LATENT] * ea[:, 0:1]
    w1 = tpw[:, LATENT:]
    out_ref[...] = jnp.concatenate(
        [w0s, w1 * ea[:, 1:2], w1 * ea[:, 2:3], w1 * ea[:, 3:4]], axis=1)


def _sc_body(x_hbm, w_hbm, send_hbm, recv_hbm, out_hbm,
             rbuf0, sbuf0, rbuf1, sbuf1, cid, csend, crecv, xg, wg, acc,
             semr0, sems0, semr1, sems1, semx, semw):
    core = lax.axis_index("c")
    tid = lax.axis_index("s")
    wid = tid * 2 + core
    iota16 = lax.iota(jnp.int32, 16)

    def start_strip(s, rb, sb, semr, sems):
        pltpu.make_async_copy(
            recv_hbm.at[pl.ds(s * STRIP, STRIP)], rb, semr).start()
        pltpu.make_async_copy(
            send_hbm.at[pl.ds(s * STRIP, STRIP)], sb, sems).start()

    def wait_strip(rb, sb, semr, sems):
        pltpu.make_async_copy(recv_hbm.at[pl.ds(0, STRIP)], rb, semr).wait()
        pltpu.make_async_copy(send_hbm.at[pl.ds(0, STRIP)], sb, sems).wait()

    def process_batch(off):
        cp1 = pltpu.async_copy(x_hbm.at[csend.at[pl.ds(off, B)]], xg, semx)
        cp2 = pltpu.async_copy(w_hbm.at[cid.at[pl.ds(off, B)]], wg, semw)
        cp1.wait()
        cp2.wait()

        def grp_body(g, _):
            rv = crecv[pl.ds(off + 16 * g, 16)]
            for j in range(16):
                e = 16 * g + j
                row = rv[j]
                for k in range(4):
                    xk = xg[e, pl.ds(16 * k, 16)]
                    for m in range(4):
                        plsc.addupdate(
                            acc.at[row, pl.ds(LATENT * m + 16 * k, 16)],
                            wg[e, pl.ds(LATENT * m + 16 * k, 16)] * xk)
            return 0
        lax.fori_loop(0, B // 16, grp_body, 0)

    def scan_strip(s, cnt, lo, rb, sb):
        sbase = s * STRIP

        def scan_body(i, cnt):
            r = rb[pl.ds(i * 16, 16)]
            rloc = r - lo
            m = (rloc >= 0) & (rloc < WINDOW)
            c16 = plsc.all_reduce_population_count(m)
            c0 = c16[0]

            @pl.when(c0 > 0)
            def _():
                sv = sb[pl.ds(i * 16, 16)]
                mi = m.astype(jnp.int32)
                pos = cnt + plsc.cumsum(mi) - 1
                eid = sbase + i * 16 + iota16
                plsc.store_scatter(cid, [pos], eid, mask=m)
                plsc.store_scatter(csend, [pos], sv, mask=m)
                plsc.store_scatter(crecv, [pos], rloc, mask=m)
            return cnt + c0
        cnt = lax.fori_loop(0, STRIP // 16, scan_body, cnt)

        nb = cnt // B

        @pl.when(nb > 0)
        def _():
            def b_body(b, _):
                process_batch(b * B)
                return 0
            lax.fori_loop(0, nb, b_body, 0)

            def mv(j, _):
                src = nb * B + j * 16
                cid[pl.ds(j * 16, 16)] = cid[pl.ds(src, 16)]
                csend[pl.ds(j * 16, 16)] = csend[pl.ds(src, 16)]
                crecv[pl.ds(j * 16, 16)] = crecv[pl.ds(src, 16)]
                return 0
            lax.fori_loop(0, B // 16, mv, 0)
        return cnt - nb * B

    def round_body(rnd, _):
        win = rnd * NWORKERS + wid
        lo = win * WINDOW

        def zr(i, _):
            for s in range(16):
                acc[i, pl.ds(16 * s, 16)] = jnp.zeros((16,), jnp.float32)
            return 0
        lax.fori_loop(0, ACCROWS, zr, 0)

        start_strip(0, rbuf0, sbuf0, semr0, sems0)

        def pair_body(p, cnt):
            start_strip(2 * p + 1, rbuf1, sbuf1, semr1, sems1)
            wait_strip(rbuf0, sbuf0, semr0, sems0)
            cnt = scan_strip(2 * p, cnt, lo, rbuf0, sbuf0)

            @pl.when(p + 1 < NSTRIPS // 2)
            def _():
                start_strip(2 * p + 2, rbuf0, sbuf0, semr0, sems0)
            wait_strip(rbuf1, sbuf1, semr1, sems1)
            cnt = scan_strip(2 * p + 1, cnt, lo, rbuf1, sbuf1)
            return cnt
        cnt = lax.fori_loop(0, NSTRIPS // 2, pair_body, jnp.int32(0))

        def padj(j, _):
            lane = j * 16 + iota16
            keep = lane < cnt
            crecv[pl.ds(j * 16, 16)] = jnp.where(
                keep, crecv[pl.ds(j * 16, 16)], WINDOW)
            csend[pl.ds(j * 16, 16)] = jnp.where(
                keep, csend[pl.ds(j * 16, 16)], iota16)
            cid[pl.ds(j * 16, 16)] = jnp.where(
                keep, cid[pl.ds(j * 16, 16)], iota16)
            return 0
        lax.fori_loop(0, B // 16, padj, 0)
        process_batch(0)

        pltpu.sync_copy(acc.at[pl.ds(0, WINDOW)],
                        out_hbm.at[pl.ds(lo, WINDOW)])
        return 0

    lax.fori_loop(0, ROUNDS, round_body, 0)


def _out_body(msg_ref, sc0_ref, wls_ref, wbig_ref, out_ref):
    msg = msg_ref[...]
    o0 = jnp.dot(msg[:, :LATENT], wls_ref[...],
                 preferred_element_type=jnp.float32) + sc0_ref[...]
    o1 = jnp.dot(msg[:, LATENT:], wbig_ref[...],
                 preferred_element_type=jnp.float32)
    out_ref[...] = jnp.concatenate([o0, o1], axis=1)


def kernel(node_attrs, node_feats, edge_attrs, edge_feats, edge_index,
           W_up, W_mlp0, W_mlp1, W_mlp2, W_mlp3, W_lin_s, W_lin_v, W_skip):
    sender = edge_index[0]
    receiver = edge_index[1]
    wsk = W_skip.reshape(LATENT, NUM_ELEM, LATENT)

    NB = 2000
    x, sc0 = pl.pallas_call(
        _node_up_body,
        grid=(N // NB,),
        in_specs=[
            pl.BlockSpec((NB, LATENT), lambda i: (i, 0)),
            pl.BlockSpec((NB, NUM_ELEM), lambda i: (i, 0)),
            pl.BlockSpec((LATENT, LATENT), lambda i: (0, 0)),
            pl.BlockSpec((LATENT, NUM_ELEM, LATENT), lambda i: (0, 0, 0)),
        ],
        out_specs=[
            pl.BlockSpec((NB, XROW), lambda i: (i, 0)),
            pl.BlockSpec((NB, LATENT), lambda i: (i, 0)),
        ],
        out_shape=[
            jax.ShapeDtypeStruct((N, XROW), jnp.float32),
            jax.ShapeDtypeStruct((N, LATENT), jnp.float32),
        ],
    )(node_feats, node_attrs, W_up, wsk)

    EB = 2000
    wfold = pl.pallas_call(
        _edge_mlp_body,
        grid=(E // EB,),
        in_specs=[
            pl.BlockSpec((EB, RADIAL), lambda i: (i, 0)),
            pl.BlockSpec((EB, 4), lambda i: (i, 0)),
            pl.BlockSpec((RADIAL, 64), lambda i: (0, 0)),
            pl.BlockSpec((64, 64), lambda i: (0, 0)),
            pl.BlockSpec((64, 64), lambda i: (0, 0)),
            pl.BlockSpec((64, 2 * LATENT), lambda i: (0, 0)),
        ],
        out_specs=pl.BlockSpec((EB, WROW), lambda i: (i, 0)),
        out_shape=jax.ShapeDtypeStruct((E, WROW), jnp.float32),
    )(edge_feats, edge_attrs, W_mlp0, W_mlp1, W_mlp2, W_mlp3)

    mesh = plsc.VectorSubcoreMesh(core_axis_name="c", subcore_axis_name="s")
    sc_call = functools.partial(
        pl.kernel,
        mesh=mesh,
        out_type=jax.ShapeDtypeStruct((OUTROWS, 4 * LATENT), jnp.float32),
        compiler_params=pltpu.CompilerParams(needs_layout_passes=False),
        scratch_types=[
            pltpu.VMEM((STRIP,), jnp.int32),
            pltpu.VMEM((STRIP,), jnp.int32),
            pltpu.VMEM((STRIP,), jnp.int32),
            pltpu.VMEM((STRIP,), jnp.int32),
            pltpu.VMEM((CAP,), jnp.int32),
            pltpu.VMEM((CAP,), jnp.int32),
            pltpu.VMEM((CAP,), jnp.int32),
            pltpu.VMEM((B, XROW), jnp.float32),
            pltpu.VMEM((B, WROW), jnp.float32),
            pltpu.VMEM((ACCROWS, WROW), jnp.float32),
            pltpu.SemaphoreType.DMA,
            pltpu.SemaphoreType.DMA,
            pltpu.SemaphoreType.DMA,
            pltpu.SemaphoreType.DMA,
            pltpu.SemaphoreType.DMA,
            pltpu.SemaphoreType.DMA,
        ],
    )(_sc_body)
    msgpad = sc_call(x, wfold, sender, receiver)

    wls = W_lin_s * INV_AVG
    wbig = ((W_lin_v[None, :, :, None] * jnp.eye(3, dtype=jnp.float32)[:, None, None, :])
            .reshape(3 * LATENT, 3 * LATENT) * INV_AVG)
    out = pl.pallas_call(
        _out_body,
        grid=(N // NB,),
        in_specs=[
            pl.BlockSpec((NB, 4 * LATENT), lambda i: (i, 0)),
            pl.BlockSpec((NB, LATENT), lambda i: (i, 0)),
            pl.BlockSpec((LATENT, LATENT), lambda i: (0, 0)),
            pl.BlockSpec((3 * LATENT, 3 * LATENT), lambda i: (0, 0)),
        ],
        out_specs=pl.BlockSpec((NB, 4 * LATENT), lambda i: (i, 0)),
        out_shape=jax.ShapeDtypeStruct((N, 4 * LATENT), jnp.float32),
    )(msgpad, sc0, wls, wbig)
    return out
